# baseline (speedup 1.0000x reference)
import functools

import jax
import jax.numpy as jnp
from jax import lax
from jax.experimental import pallas as pl
from jax.experimental.pallas import tpu as pltpu
from jax.experimental.pallas import tpu_sc as plsc

TILE = 64
PAD_ROWS = 768

_SC_CORES = 2
_SC_SUBCORES = 16
_NW = _SC_CORES * _SC_SUBCORES
_SC_BUF_BYTES = 384 * 1024


def _router_body(x_ref, gw_ref, logits_ref, eid_ref):
    x = x_ref[...]
    logits = lax.dot_general(x, gw_ref[...], (((1,), (1,)), ((), ())),
                             preferred_element_type=jnp.float32)
    logits_ref[...] = logits
    eid_ref[...] = jnp.argmax(logits, axis=1, keepdims=True).astype(jnp.int32)


def _router(flat, gate_w):
    s, _ = flat.shape
    e, _ = gate_w.shape
    return pl.pallas_call(
        _router_body,
        out_shape=(
            jax.ShapeDtypeStruct((s, e), jnp.float32),
            jax.ShapeDtypeStruct((s, 1), jnp.int32),
        ),
    )(flat, gate_w)


def _ffn_body(e_ref, base_ref, xs_ref, wg_ref, wu_ref, wd_ref, out_ref):
    t = pl.program_id(0)
    base = pl.multiple_of(base_ref[t], 8)
    x = xs_ref[pl.ds(base, TILE), :]
    dn = (((1,), (1,)), ((), ()))
    h1 = lax.dot_general(x, wg_ref[0], dn, preferred_element_type=jnp.float32)
    h2 = lax.dot_general(x, wu_ref[0], dn, preferred_element_type=jnp.float32)
    h = (h1 * jax.nn.sigmoid(h1)) * h2
    out_ref[pl.ds(base, TILE), :] = lax.dot_general(
        h, wd_ref[0], dn, preferred_element_type=jnp.float32)


def _grouped_ffn(x_rows, Wg, Wu, Wd, tile_e, tile_start, nt):
    p, d = x_rows.shape
    _, i, _ = Wg.shape
    grid_spec = pltpu.PrefetchScalarGridSpec(
        num_scalar_prefetch=2,
        grid=(nt,),
        in_specs=[
            pl.BlockSpec((p, d), lambda t, e, b: (0, 0)),
            pl.BlockSpec((1, i, d), lambda t, e, b: (e[t], 0, 0)),
            pl.BlockSpec((1, i, d), lambda t, e, b: (e[t], 0, 0)),
            pl.BlockSpec((1, d, i), lambda t, e, b: (e[t], 0, 0)),
        ],
        out_specs=pl.BlockSpec((p, d), lambda t, e, b: (0, 0)),
    )
    return pl.pallas_call(
        _ffn_body,
        grid_spec=grid_spec,
        out_shape=jax.ShapeDtypeStruct((p, d), jnp.float32),
        compiler_params=pltpu.CompilerParams(
            dimension_semantics=("arbitrary",)),
    )(tile_e, tile_start, x_rows, Wg, Wu, Wd)


def _sc_gather_rows(table, idx):
    _, d = table.shape
    n = idx.shape[0]
    b_per_w = n // _NW
    n_chunks = 1
    while (b_per_w // n_chunks) * d * 4 > _SC_BUF_BYTES:
        n_chunks *= 2
    chunk = b_per_w // n_chunks
    mesh = plsc.VectorSubcoreMesh(
        core_axis_name="c", subcore_axis_name="s",
        num_cores=_SC_CORES, num_subcores=_SC_SUBCORES)

    @functools.partial(
        pl.kernel,
        out_type=jax.ShapeDtypeStruct((n, d), jnp.float32),
        mesh=mesh,
        scratch_types=[
            pltpu.VMEM((chunk,), jnp.int32),
            pltpu.VMEM((chunk, d), jnp.float32),
            pltpu.SemaphoreType.DMA,
        ],
    )
    def k(table_hbm, idx_hbm, out_hbm, idx_v, rows_v, sem):
        wid = lax.axis_index("s") * _SC_CORES + lax.axis_index("c")
        base = wid * b_per_w
        for c in range(n_chunks):
            off = base + c * chunk
            pltpu.sync_copy(idx_hbm.at[pl.ds(off, chunk)], idx_v)
            pltpu.async_copy(table_hbm.at[idx_v], rows_v, sem).wait()
            pltpu.sync_copy(rows_v, out_hbm.at[pl.ds(off, chunk)])

    return k(table, idx)


def kernel(hidden_states, gate_w, Wg, Wu, Wd):
    bsz, seq_len, d = hidden_states.shape
    e = gate_w.shape[0]
    flat = hidden_states.reshape(-1, d)
    s = flat.shape[0]
    p = s + PAD_ROWS

    logits, eids = _router(flat, gate_w)
    eid = eids[:, 0]

    sorted_eid, perm = lax.sort((eid, jnp.arange(s, dtype=jnp.int32)),
                                dimension=0, num_keys=1, is_stable=True)
    counts = jnp.bincount(eid, length=e).astype(jnp.int32)
    starts = jnp.concatenate(
        [jnp.zeros((1,), jnp.int32), jnp.cumsum(counts)[:-1].astype(jnp.int32)])
    acounts = (counts + 7) & ~7
    astarts = jnp.concatenate(
        [jnp.zeros((1,), jnp.int32),
         jnp.cumsum(acounts)[:-1].astype(jnp.int32)])

    tiles_per_e = (counts + TILE - 1) // TILE
    nt = s // TILE + e
    tile_e = jnp.repeat(jnp.arange(e, dtype=jnp.int32), tiles_per_e,
                        total_repeat_length=nt)
    tile_base = jnp.concatenate(
        [jnp.zeros((1,), jnp.int32),
         jnp.cumsum(tiles_per_e)[:-1].astype(jnp.int32)])
    tile_k = jnp.arange(nt, dtype=jnp.int32) - tile_base[tile_e]
    tile_start = jnp.clip(astarts[tile_e] + tile_k * TILE, 0, p - TILE)

    sorted_pos = jnp.arange(s, dtype=jnp.int32)
    slot_of_sorted = astarts[sorted_eid] + (sorted_pos - starts[sorted_eid])
    slot_token = jnp.zeros((p,), jnp.int32).at[slot_of_sorted].set(perm)
    token_slot = jnp.zeros((s,), jnp.int32).at[perm].set(slot_of_sorted)

    x_rows = _sc_gather_rows(flat, slot_token)
    out_rows = _grouped_ffn(x_rows, Wg, Wu, Wd, tile_e, tile_start, nt)
    out_flat = _sc_gather_rows(out_rows, token_slot)

    return out_flat.reshape(bsz, seq_len, d), logits


# speedup vs baseline: 1.6625x; 1.6625x over previous
import functools

import jax
import jax.numpy as jnp
from jax import lax
from jax.experimental import pallas as pl
from jax.experimental.pallas import tpu as pltpu
from jax.experimental.pallas import tpu_sc as plsc

TILE = 128
BLK = 128
PAD_ROWS = 768

_SC_CORES = 2
_SC_SUBCORES = 16
_NW = _SC_CORES * _SC_SUBCORES
_SC_BUF_BYTES = 384 * 1024


def _router_body(x_ref, gw_ref, logits_ref, slot_ref, tile_e_ref,
                 tile_start_ref):
    s = x_ref.shape[0]
    e = gw_ref.shape[0]
    nt = tile_e_ref.shape[0]
    p = s + PAD_ROWS
    nblk = s // BLK
    dn = (((1,), (1,)), ((), ()))
    dn2 = (((1,), (0,)), ((), ()))

    logits = lax.dot_general(x_ref[...], gw_ref[...], dn,
                             preferred_element_type=jnp.float32)
    logits_ref[...] = logits

    is_max = (logits == jnp.max(logits, axis=1, keepdims=True)).astype(
        jnp.float32)
    ut_e = (lax.broadcasted_iota(jnp.int32, (e, e), 0)
            <= lax.broadcasted_iota(jnp.int32, (e, e), 1)).astype(jnp.float32)
    first = lax.dot_general(is_max, ut_e, dn2, preferred_element_type=jnp.float32)
    oh = jnp.where((is_max > 0.0) & (first == 1.0), 1.0, 0.0)

    oh3 = oh.reshape(nblk, BLK, e)
    blockcounts = jnp.sum(oh3, axis=1)
    lt_b = (lax.broadcasted_iota(jnp.int32, (nblk, nblk), 1)
            < lax.broadcasted_iota(jnp.int32, (nblk, nblk), 0)).astype(
                jnp.float32)
    blockprefix = lax.dot_general(lt_b, blockcounts, dn2,
                                  preferred_element_type=jnp.float32)
    counts = jnp.sum(blockcounts, axis=0)

    acounts = jnp.floor((counts + 7.0) * 0.125) * 8.0
    lt_e = (lax.broadcasted_iota(jnp.int32, (e, e), 1)
            < lax.broadcasted_iota(jnp.int32, (e, e), 0)).astype(jnp.float32)
    astarts = lax.dot_general(lt_e, acounts.reshape(e, 1), dn2,
                              preferred_element_type=jnp.float32)[:, 0]

    lt_t = (lax.broadcasted_iota(jnp.int32, (BLK, BLK), 1)
            < lax.broadcasted_iota(jnp.int32, (BLK, BLK), 0)).astype(
                jnp.float32)
    for b in range(nblk):
        oh_b = oh3[b]
        rank_b = jnp.sum(
            lax.dot_general(lt_t, oh_b, dn2, preferred_element_type=jnp.float32)
            * oh_b, axis=1)
        base_b = jnp.sum(oh_b * (astarts + blockprefix[b])[None, :], axis=1)
        slot_ref[b * BLK:(b + 1) * BLK, 0] = (rank_b + base_b).astype(jnp.int32)

    tiles_per_e = jnp.floor((counts + (TILE - 1.0)) * (1.0 / TILE))
    cum_excl = lax.dot_general(lt_e, tiles_per_e.reshape(e, 1), dn2,
                               preferred_element_type=jnp.float32)[:, 0]
    cum_incl = cum_excl + tiles_per_e
    i_idx = lax.broadcasted_iota(jnp.int32, (nt, e), 0).astype(jnp.float32)
    tile_e = jnp.minimum(jnp.sum(
        jnp.where(cum_incl[None, :] <= i_idx, 1.0, 0.0), axis=1),
        float(e - 1))
    oh_te = jnp.where(
        lax.broadcasted_iota(jnp.int32, (nt, e), 1).astype(jnp.float32)
        == tile_e[:, None], 1.0, 0.0)
    astart_t = jnp.sum(oh_te * astarts[None, :], axis=1)
    cumex_t = jnp.sum(oh_te * cum_excl[None, :], axis=1)
    tile_i = lax.broadcasted_iota(jnp.int32, (nt, 1), 0).astype(
        jnp.float32)[:, 0]
    tile_start = jnp.clip(astart_t + (tile_i - cumex_t) * TILE,
                          0.0, float(p - TILE))
    tile_e_ref[:, 0] = tile_e.astype(jnp.int32)
    tile_start_ref[:, 0] = tile_start.astype(jnp.int32)


def _router(flat, gate_w, nt):
    s, _ = flat.shape
    e = gate_w.shape[0]
    return pl.pallas_call(
        _router_body,
        out_shape=(
            jax.ShapeDtypeStruct((s, e), jnp.float32),
            jax.ShapeDtypeStruct((s, 1), jnp.int32),
            jax.ShapeDtypeStruct((nt, 1), jnp.int32),
            jax.ShapeDtypeStruct((nt, 1), jnp.int32),
        ),
    )(flat, gate_w)


def _ffn_body(e_ref, base_ref, xs_ref, wg_ref, wu_ref, wd_ref, out_ref):
    t = pl.program_id(0)
    base = pl.multiple_of(base_ref[t], 8)
    x = xs_ref[pl.ds(base, TILE), :]
    dn = (((1,), (1,)), ((), ()))
    h1 = lax.dot_general(x, wg_ref[0], dn, preferred_element_type=jnp.float32)
    h2 = lax.dot_general(x, wu_ref[0], dn, preferred_element_type=jnp.float32)
    h = (h1 * jax.nn.sigmoid(h1)) * h2
    out_ref[pl.ds(base, TILE), :] = lax.dot_general(
        h, wd_ref[0], dn, preferred_element_type=jnp.float32)


def _grouped_ffn(x_rows, Wg, Wu, Wd, tile_e, tile_start, nt):
    p, d = x_rows.shape
    _, i, _ = Wg.shape
    grid_spec = pltpu.PrefetchScalarGridSpec(
        num_scalar_prefetch=2,
        grid=(nt,),
        in_specs=[
            pl.BlockSpec((p, d), lambda t, e, b: (0, 0)),
            pl.BlockSpec((1, i, d), lambda t, e, b: (e[t], 0, 0)),
            pl.BlockSpec((1, i, d), lambda t, e, b: (e[t], 0, 0)),
            pl.BlockSpec((1, d, i), lambda t, e, b: (e[t], 0, 0)),
        ],
        out_specs=pl.BlockSpec((p, d), lambda t, e, b: (0, 0)),
    )
    return pl.pallas_call(
        _ffn_body,
        grid_spec=grid_spec,
        out_shape=jax.ShapeDtypeStruct((p, d), jnp.float32),
        compiler_params=pltpu.CompilerParams(
            dimension_semantics=("arbitrary",)),
    )(tile_e, tile_start, x_rows, Wg, Wu, Wd)


def _sc_mesh():
    return plsc.VectorSubcoreMesh(
        core_axis_name="c", subcore_axis_name="s",
        num_cores=_SC_CORES, num_subcores=_SC_SUBCORES)


def _sc_scatter_rows(rows, idx, p):
    n, d = rows.shape
    b_per_w = n // _NW

    @functools.partial(
        pl.kernel,
        out_type=jax.ShapeDtypeStruct((p, d), jnp.float32),
        mesh=_sc_mesh(),
        scratch_types=[
            pltpu.VMEM((b_per_w,), jnp.int32),
            pltpu.VMEM((b_per_w, d), jnp.float32),
            pltpu.SemaphoreType.DMA,
        ],
    )
    def k(rows_hbm, idx_hbm, out_hbm, idx_v, rows_v, sem):
        wid = lax.axis_index("s") * _SC_CORES + lax.axis_index("c")
        base = wid * b_per_w
        pltpu.sync_copy(idx_hbm.at[pl.ds(base, b_per_w)], idx_v)
        pltpu.sync_copy(rows_hbm.at[pl.ds(base, b_per_w)], rows_v)
        pltpu.async_copy(rows_v, out_hbm.at[idx_v], sem).wait()

    return k(rows, idx)


def _sc_gather_rows(table, idx):
    _, d = table.shape
    n = idx.shape[0]
    b_per_w = n // _NW
    n_chunks = 1
    while (b_per_w // n_chunks) * d * 4 > _SC_BUF_BYTES:
        n_chunks *= 2
    chunk = b_per_w // n_chunks

    @functools.partial(
        pl.kernel,
        out_type=jax.ShapeDtypeStruct((n, d), jnp.float32),
        mesh=_sc_mesh(),
        scratch_types=[
            pltpu.VMEM((chunk,), jnp.int32),
            pltpu.VMEM((chunk, d), jnp.float32),
            pltpu.SemaphoreType.DMA,
        ],
    )
    def k(table_hbm, idx_hbm, out_hbm, idx_v, rows_v, sem):
        wid = lax.axis_index("s") * _SC_CORES + lax.axis_index("c")
        base = wid * b_per_w
        for c in range(n_chunks):
            off = base + c * chunk
            pltpu.sync_copy(idx_hbm.at[pl.ds(off, chunk)], idx_v)
            pltpu.async_copy(table_hbm.at[idx_v], rows_v, sem).wait()
            pltpu.sync_copy(rows_v, out_hbm.at[pl.ds(off, chunk)])

    return k(table, idx)


def kernel(hidden_states, gate_w, Wg, Wu, Wd):
    bsz, seq_len, d = hidden_states.shape
    e = gate_w.shape[0]
    flat = hidden_states.reshape(-1, d)
    s = flat.shape[0]
    p = s + PAD_ROWS
    nt = s // TILE + e

    logits, token_slot, tile_e, tile_start = _router(flat, gate_w, nt)

    x_rows = _sc_scatter_rows(flat, token_slot[:, 0], p)
    out_rows = _grouped_ffn(x_rows, Wg, Wu, Wd,
                            tile_e[:, 0], tile_start[:, 0], nt)
    out_flat = _sc_gather_rows(out_rows, token_slot[:, 0])

    return out_flat.reshape(bsz, seq_len, d), logits


# speedup vs baseline: 1.8563x; 1.1166x over previous
import functools

import jax
import jax.numpy as jnp
from jax import lax
from jax.experimental import pallas as pl
from jax.experimental.pallas import tpu as pltpu
from jax.experimental.pallas import tpu_sc as plsc

TILE = 128
BLK = 128
PAD_ROWS = 768

_SC_CORES = 2
_SC_SUBCORES = 16
_NW = _SC_CORES * _SC_SUBCORES
_SC_BUF_BYTES = 384 * 1024


def _router_body(x_ref, gw_ref, logits_ref, slot_ref, tile_e_ref,
                 tile_start_ref, nreal_ref):
    s = x_ref.shape[0]
    e = gw_ref.shape[0]
    nt = tile_e_ref.shape[0]
    p = s + PAD_ROWS
    nblk = s // BLK
    dn = (((1,), (1,)), ((), ()))
    dn2 = (((1,), (0,)), ((), ()))

    logits = lax.dot_general(x_ref[...], gw_ref[...], dn,
                             preferred_element_type=jnp.float32)
    logits_ref[...] = logits

    is_max = (logits == jnp.max(logits, axis=1, keepdims=True)).astype(
        jnp.float32)
    ut_e = (lax.broadcasted_iota(jnp.int32, (e, e), 0)
            <= lax.broadcasted_iota(jnp.int32, (e, e), 1)).astype(jnp.float32)
    first = lax.dot_general(is_max, ut_e, dn2, preferred_element_type=jnp.float32)
    oh = jnp.where((is_max > 0.0) & (first == 1.0), 1.0, 0.0)

    oh3 = oh.reshape(nblk, BLK, e)
    blockcounts = jnp.sum(oh3, axis=1)
    lt_b = (lax.broadcasted_iota(jnp.int32, (nblk, nblk), 1)
            < lax.broadcasted_iota(jnp.int32, (nblk, nblk), 0)).astype(
                jnp.float32)
    blockprefix = lax.dot_general(lt_b, blockcounts, dn2,
                                  preferred_element_type=jnp.float32)
    counts = jnp.sum(blockcounts, axis=0)

    acounts = jnp.floor((counts + 7.0) * 0.125) * 8.0
    lt_e = (lax.broadcasted_iota(jnp.int32, (e, e), 1)
            < lax.broadcasted_iota(jnp.int32, (e, e), 0)).astype(jnp.float32)
    astarts = lax.dot_general(lt_e, acounts.reshape(e, 1), dn2,
                              preferred_element_type=jnp.float32)[:, 0]

    lt_t = (lax.broadcasted_iota(jnp.int32, (BLK, BLK), 1)
            < lax.broadcasted_iota(jnp.int32, (BLK, BLK), 0)).astype(
                jnp.float32)
    for b in range(nblk):
        oh_b = oh3[b]
        rank_b = jnp.sum(
            lax.dot_general(lt_t, oh_b, dn2, preferred_element_type=jnp.float32)
            * oh_b, axis=1)
        base_b = jnp.sum(oh_b * (astarts + blockprefix[b])[None, :], axis=1)
        slot_ref[b * BLK:(b + 1) * BLK, 0] = (rank_b + base_b).astype(jnp.int32)

    tiles_per_e = jnp.floor((counts + (TILE - 1.0)) * (1.0 / TILE))
    cum_excl = lax.dot_general(lt_e, tiles_per_e.reshape(e, 1), dn2,
                               preferred_element_type=jnp.float32)[:, 0]
    cum_incl = cum_excl + tiles_per_e
    i_idx = lax.broadcasted_iota(jnp.int32, (nt, e), 0).astype(jnp.float32)
    tile_e = jnp.minimum(jnp.sum(
        jnp.where(cum_incl[None, :] <= i_idx, 1.0, 0.0), axis=1),
        float(e - 1))
    n_real = jnp.sum(tiles_per_e)
    e_ids = lax.broadcasted_iota(jnp.int32, (e, 1), 0).astype(jnp.float32)[:, 0]
    last_e = jnp.max(jnp.where(counts > 0.0, e_ids, 0.0))
    tile_i0 = lax.broadcasted_iota(jnp.int32, (nt, 1), 0).astype(
        jnp.float32)[:, 0]
    tile_e = jnp.where(tile_i0 < n_real, tile_e, last_e)
    oh_te = jnp.where(
        lax.broadcasted_iota(jnp.int32, (nt, e), 1).astype(jnp.float32)
        == tile_e[:, None], 1.0, 0.0)
    astart_t = jnp.sum(oh_te * astarts[None, :], axis=1)
    cumex_t = jnp.sum(oh_te * cum_excl[None, :], axis=1)
    tile_i = lax.broadcasted_iota(jnp.int32, (nt, 1), 0).astype(
        jnp.float32)[:, 0]
    tile_start = jnp.clip(astart_t + (tile_i - cumex_t) * TILE,
                          0.0, float(p - TILE))
    tile_e_ref[:, 0] = tile_e.astype(jnp.int32)
    tile_start_ref[:, 0] = tile_start.astype(jnp.int32)
    nreal_ref[...] = jnp.full((8, 128), n_real, jnp.float32).astype(jnp.int32)


def _router(flat, gate_w, nt):
    s, _ = flat.shape
    e = gate_w.shape[0]
    return pl.pallas_call(
        _router_body,
        out_shape=(
            jax.ShapeDtypeStruct((s, e), jnp.float32),
            jax.ShapeDtypeStruct((s, 1), jnp.int32),
            jax.ShapeDtypeStruct((nt, 1), jnp.int32),
            jax.ShapeDtypeStruct((nt, 1), jnp.int32),
            jax.ShapeDtypeStruct((8, 128), jnp.int32),
        ),
    )(flat, gate_w)


def _ffn_body(e_ref, base_ref, nreal_ref, xs_ref, wg_ref, wu_ref, wd_ref,
              out_ref):
    t = pl.program_id(0)

    @pl.when(t < nreal_ref[0])
    def _():
        base = pl.multiple_of(base_ref[t], 8)
        x = xs_ref[pl.ds(base, TILE), :]
        dn = (((1,), (1,)), ((), ()))
        h1 = lax.dot_general(x, wg_ref[0], dn,
                             preferred_element_type=jnp.float32)
        h2 = lax.dot_general(x, wu_ref[0], dn,
                             preferred_element_type=jnp.float32)
        h = (h1 * jax.nn.sigmoid(h1)) * h2
        out_ref[pl.ds(base, TILE), :] = lax.dot_general(
            h, wd_ref[0], dn, preferred_element_type=jnp.float32)


def _grouped_ffn(x_rows, Wg, Wu, Wd, tile_e, tile_start, n_real, nt):
    p, d = x_rows.shape
    _, i, _ = Wg.shape
    grid_spec = pltpu.PrefetchScalarGridSpec(
        num_scalar_prefetch=3,
        grid=(nt,),
        in_specs=[
            pl.BlockSpec((p, d), lambda t, e, b, nr: (0, 0)),
            pl.BlockSpec((1, i, d), lambda t, e, b, nr: (e[t], 0, 0)),
            pl.BlockSpec((1, i, d), lambda t, e, b, nr: (e[t], 0, 0)),
            pl.BlockSpec((1, d, i), lambda t, e, b, nr: (e[t], 0, 0)),
        ],
        out_specs=pl.BlockSpec((p, d), lambda t, e, b, nr: (0, 0)),
    )
    return pl.pallas_call(
        _ffn_body,
        grid_spec=grid_spec,
        out_shape=jax.ShapeDtypeStruct((p, d), jnp.float32),
        compiler_params=pltpu.CompilerParams(
            dimension_semantics=("arbitrary",)),
    )(tile_e, tile_start, n_real, x_rows, Wg, Wu, Wd)


def _sc_mesh():
    return plsc.VectorSubcoreMesh(
        core_axis_name="c", subcore_axis_name="s",
        num_cores=_SC_CORES, num_subcores=_SC_SUBCORES)


def _sc_scatter_rows(rows, idx, p):
    n, d = rows.shape
    b_per_w = n // _NW

    @functools.partial(
        pl.kernel,
        out_type=jax.ShapeDtypeStruct((p, d), jnp.float32),
        mesh=_sc_mesh(),
        scratch_types=[
            pltpu.VMEM((b_per_w,), jnp.int32),
            pltpu.VMEM((b_per_w, d), jnp.float32),
            pltpu.SemaphoreType.DMA,
        ],
    )
    def k(rows_hbm, idx_hbm, out_hbm, idx_v, rows_v, sem):
        wid = lax.axis_index("s") * _SC_CORES + lax.axis_index("c")
        base = wid * b_per_w
        pltpu.sync_copy(idx_hbm.at[pl.ds(base, b_per_w)], idx_v)
        pltpu.sync_copy(rows_hbm.at[pl.ds(base, b_per_w)], rows_v)
        pltpu.async_copy(rows_v, out_hbm.at[idx_v], sem).wait()

    return k(rows, idx)


def _sc_gather_rows(table, idx):
    _, d = table.shape
    n = idx.shape[0]
    b_per_w = n // _NW
    n_chunks = 1
    while (b_per_w // n_chunks) * d * 4 > _SC_BUF_BYTES:
        n_chunks *= 2
    chunk = b_per_w // n_chunks

    @functools.partial(
        pl.kernel,
        out_type=jax.ShapeDtypeStruct((n, d), jnp.float32),
        mesh=_sc_mesh(),
        scratch_types=[
            pltpu.VMEM((chunk,), jnp.int32),
            pltpu.VMEM((chunk, d), jnp.float32),
            pltpu.SemaphoreType.DMA,
        ],
    )
    def k(table_hbm, idx_hbm, out_hbm, idx_v, rows_v, sem):
        wid = lax.axis_index("s") * _SC_CORES + lax.axis_index("c")
        base = wid * b_per_w
        for c in range(n_chunks):
            off = base + c * chunk
            pltpu.sync_copy(idx_hbm.at[pl.ds(off, chunk)], idx_v)
            pltpu.async_copy(table_hbm.at[idx_v], rows_v, sem).wait()
            pltpu.sync_copy(rows_v, out_hbm.at[pl.ds(off, chunk)])

    return k(table, idx)


def kernel(hidden_states, gate_w, Wg, Wu, Wd):
    bsz, seq_len, d = hidden_states.shape
    e = gate_w.shape[0]
    flat = hidden_states.reshape(-1, d)
    s = flat.shape[0]
    p = s + PAD_ROWS
    nt = s // TILE + e

    logits, token_slot, tile_e, tile_start, n_real = _router(flat, gate_w, nt)

    x_rows = _sc_scatter_rows(flat, token_slot[:, 0], p)
    out_rows = _grouped_ffn(x_rows, Wg, Wu, Wd,
                            tile_e[:, 0], tile_start[:, 0], n_real[0, 0:1], nt)
    out_flat = _sc_gather_rows(out_rows, token_slot[:, 0])

    return out_flat.reshape(bsz, seq_len, d), logits
